# Initial kernel scaffold; baseline (speedup 1.0000x reference)
import functools

import jax
import jax.numpy as jnp
from jax import lax
from jax.experimental import pallas as pl
from jax.experimental.pallas import tpu as pltpu
from jax.experimental.pallas import tpu_sc as plsc

DIM = 64
NC = 2
NS = 16
NW = NC * NS
CHUNK = 128


@functools.cache
def _make_kernel(B: int):
    bpw = B // NW
    nch = bpw // CHUNK
    mesh = plsc.VectorSubcoreMesh(core_axis_name="c", subcore_axis_name="s")

    @functools.partial(
        pl.kernel,
        mesh=mesh,
        out_type=jax.ShapeDtypeStruct((NW, nch, CHUNK, DIM), jnp.float32),
        scratch_types=[
            pltpu.VMEM((nch, CHUNK), jnp.int32),
            pltpu.VMEM((CHUNK, DIM), jnp.float32),
            pltpu.SemaphoreType.DMA,
        ],
    )
    def emb(tok_hbm, table_hbm, out_hbm, idx_v, rows_v, sem):
        wid = lax.axis_index("s") * NC + lax.axis_index("c")
        pltpu.sync_copy(tok_hbm.at[wid], idx_v)

        def body(j, carry):
            pltpu.async_copy(table_hbm.at[idx_v.at[j]], rows_v, sem).wait()
            pltpu.sync_copy(rows_v, out_hbm.at[wid, j])
            return carry

        lax.fori_loop(0, nch, body, 0)

    return emb


def kernel(tokens, embedding):
    n, s = tokens.shape
    B = n * s
    tok = tokens.reshape(NW, B // NW // CHUNK, CHUNK).astype(jnp.int32)
    out = _make_kernel(B)(tok, embedding)
    return out.reshape(n, s, DIM)


# speedup vs baseline: 1.6855x; 1.6855x over previous
import functools

import jax
import jax.numpy as jnp
from jax import lax
from jax.experimental import pallas as pl
from jax.experimental.pallas import tpu as pltpu
from jax.experimental.pallas import tpu_sc as plsc

DIM = 64
NC = 2
NS = 16
NW = NC * NS
CHUNK = 128


@functools.cache
def _make_kernel(B: int):
    bpw = B // NW
    nch = bpw // CHUNK
    mesh = plsc.VectorSubcoreMesh(core_axis_name="c", subcore_axis_name="s")

    @functools.partial(
        pl.kernel,
        mesh=mesh,
        out_type=jax.ShapeDtypeStruct((NW, nch, CHUNK, DIM), jnp.float32),
        scratch_types=[
            pltpu.VMEM((nch, CHUNK), jnp.int32),
            pltpu.VMEM((CHUNK, DIM), jnp.float32),
            pltpu.SemaphoreType.DMA,
        ],
        compiler_params=pltpu.CompilerParams(use_tc_tiling_on_sc=False),
    )
    def emb(tok_hbm, table_hbm, out_hbm, idx_v, rows_v, sem):
        wid = lax.axis_index("s") * NC + lax.axis_index("c")
        pltpu.sync_copy(tok_hbm.at[wid], idx_v)

        def body(j, carry):
            pltpu.async_copy(table_hbm.at[idx_v.at[j]], rows_v, sem).wait()
            pltpu.sync_copy(rows_v, out_hbm.at[wid, j])
            return carry

        lax.fori_loop(0, nch, body, 0)

    return emb


def kernel(tokens, embedding):
    n, s = tokens.shape
    B = n * s
    tok = tokens.reshape(NW, B // NW // CHUNK, CHUNK).astype(jnp.int32)
    out = _make_kernel(B)(tok, embedding)
    return out.reshape(n, s, DIM)


# speedup vs baseline: 1.8875x; 1.1198x over previous
import functools

import jax
import jax.numpy as jnp
from jax import lax
from jax.experimental import pallas as pl
from jax.experimental.pallas import tpu as pltpu
from jax.experimental.pallas import tpu_sc as plsc

DIM = 64
NC = 2
NS = 16
NW = NC * NS
CHUNK = 128
NBUF = 4


@functools.cache
def _make_kernel(B: int):
    bpw = B // NW
    nch = bpw // CHUNK
    ngrp = nch // (2 * NBUF)
    assert nch % (2 * NBUF) == 0
    mesh = plsc.VectorSubcoreMesh(core_axis_name="c", subcore_axis_name="s")

    @functools.partial(
        pl.kernel,
        mesh=mesh,
        out_type=jax.ShapeDtypeStruct((NW, nch, CHUNK, DIM), jnp.float32),
        scratch_types=[
            pltpu.VMEM((nch, CHUNK), jnp.int32),
            pltpu.VMEM((2 * NBUF, CHUNK, DIM), jnp.float32),
            pltpu.SemaphoreType.DMA,
            pltpu.SemaphoreType.DMA,
            pltpu.SemaphoreType.DMA,
            pltpu.SemaphoreType.DMA,
        ],
        compiler_params=pltpu.CompilerParams(use_tc_tiling_on_sc=False),
    )
    def emb(tok_hbm, table_hbm, out_hbm, idx_v, rows_v, g0, g1, w0, w1):
        wid = lax.axis_index("s") * NC + lax.axis_index("c")
        pltpu.sync_copy(tok_hbm.at[wid], idx_v)

        def fire_gather(chunk, buf, sem):
            pltpu.async_copy(table_hbm.at[idx_v.at[chunk]], rows_v.at[buf], sem)

        def drain(sem):
            pltpu.make_async_copy(out_hbm.at[wid, 0], rows_v.at[0], sem).wait()

        for b in range(NBUF):
            fire_gather(b, b, g0)

        def body(h, carry):
            base = h * (2 * NBUF)

            @pl.when(h > 0)
            def _():
                for b in range(NBUF):
                    drain(w1)

            for b in range(NBUF):
                fire_gather(base + NBUF + b, NBUF + b, g1)

            for b in range(NBUF):
                drain(g0)
            for b in range(NBUF):
                pltpu.async_copy(rows_v.at[b], out_hbm.at[wid, base + b], w0)

            for b in range(NBUF):
                drain(w0)

            @pl.when(h + 1 < ngrp)
            def _():
                for b in range(NBUF):
                    fire_gather(base + 2 * NBUF + b, b, g0)

            for b in range(NBUF):
                drain(g1)
            for b in range(NBUF):
                pltpu.async_copy(rows_v.at[NBUF + b], out_hbm.at[wid, base + NBUF + b], w1)
            return carry

        lax.fori_loop(0, ngrp, body, 0)
        for b in range(NBUF):
            drain(w1)

    return emb


def kernel(tokens, embedding):
    n, s = tokens.shape
    B = n * s
    tok = tokens.reshape(NW, B // NW // CHUNK, CHUNK).astype(jnp.int32)
    out = _make_kernel(B)(tok, embedding)
    return out.reshape(n, s, DIM)


# speedup vs baseline: 2.5451x; 1.3484x over previous
import functools

import jax
import jax.numpy as jnp
from jax import lax
from jax.experimental import pallas as pl
from jax.experimental.pallas import tpu as pltpu
from jax.experimental.pallas import tpu_sc as plsc

DIM = 64
NC = 2
NS = 16
NW = NC * NS
CHUNK = 128
NBUF = 2


@functools.cache
def _tc_pack_table(V: int):
    TB = 4096

    def body(in_ref, out_ref):
        out_ref[:, 0:DIM] = in_ref[...].T

    return pl.pallas_call(
        body,
        grid=((V + TB - 1) // TB,),
        in_specs=[pl.BlockSpec((DIM, TB), lambda i: (0, i))],
        out_specs=pl.BlockSpec((TB, 2 * DIM), lambda i: (i, 0)),
        out_shape=jax.ShapeDtypeStruct((V, 2 * DIM), jnp.float32),
    )


@functools.cache
def _tc_transpose_out(N: int, SD: int):
    NB = 256

    def body(in_ref, out_ref):
        out_ref[...] = in_ref[...].T

    return pl.pallas_call(
        body,
        grid=(N // NB,),
        in_specs=[pl.BlockSpec((NB, SD), lambda i: (i, 0))],
        out_specs=pl.BlockSpec((SD, NB), lambda i: (0, i)),
        out_shape=jax.ShapeDtypeStruct((SD, N), jnp.float32),
    )


@functools.cache
def _sc_gather(B: int, V: int):
    bpw = B // NW
    nch = bpw // CHUNK
    ngrp = nch // (2 * NBUF)
    assert nch % (2 * NBUF) == 0
    mesh = plsc.VectorSubcoreMesh(core_axis_name="c", subcore_axis_name="s")

    @functools.partial(
        pl.kernel,
        mesh=mesh,
        out_type=jax.ShapeDtypeStruct((B, DIM), jnp.float32),
        scratch_types=[
            pltpu.VMEM((nch, CHUNK), jnp.int32),
            pltpu.VMEM((2 * NBUF, CHUNK, 2 * DIM), jnp.float32),
            pltpu.SemaphoreType.DMA,
            pltpu.SemaphoreType.DMA,
            pltpu.SemaphoreType.DMA,
            pltpu.SemaphoreType.DMA,
        ],
        compiler_params=pltpu.CompilerParams(use_tc_tiling_on_sc=False),
    )
    def emb(tok_hbm, table_hbm, out_hbm, idx_v, rows_v, g0, g1, w0, w1):
        wid = lax.axis_index("s") * NC + lax.axis_index("c")
        base_row = wid * bpw
        pltpu.sync_copy(tok_hbm.at[wid], idx_v)

        def fire_gather(chunk, buf, sem):
            pltpu.async_copy(table_hbm.at[idx_v.at[chunk]], rows_v.at[buf], sem)

        def fire_write(chunk, buf, sem):
            pltpu.async_copy(
                rows_v.at[buf, :, pl.ds(0, DIM)],
                out_hbm.at[pl.ds(base_row + chunk * CHUNK, CHUNK)],
                sem,
            )

        def drain_g(sem):
            pltpu.make_async_copy(
                table_hbm.at[pl.ds(0, CHUNK)], rows_v.at[0], sem
            ).wait()

        def drain_w(sem):
            pltpu.make_async_copy(
                rows_v.at[0, :, pl.ds(0, DIM)],
                out_hbm.at[pl.ds(base_row, CHUNK)],
                sem,
            ).wait()

        for b in range(NBUF):
            fire_gather(b, b, g0)

        def body(h, carry):
            base = h * (2 * NBUF)

            @pl.when(h > 0)
            def _():
                for b in range(NBUF):
                    drain_w(w1)

            for b in range(NBUF):
                fire_gather(base + NBUF + b, NBUF + b, g1)

            for b in range(NBUF):
                drain_g(g0)
            for b in range(NBUF):
                fire_write(base + b, b, w0)

            for b in range(NBUF):
                drain_w(w0)

            @pl.when(h + 1 < ngrp)
            def _():
                for b in range(NBUF):
                    fire_gather(base + 2 * NBUF + b, b, g0)

            for b in range(NBUF):
                drain_g(g1)
            for b in range(NBUF):
                fire_write(base + NBUF + b, NBUF + b, w1)
            return carry

        lax.fori_loop(0, ngrp, body, 0)
        for b in range(NBUF):
            drain_w(w1)

    return emb


def kernel(tokens, embedding):
    n, s = tokens.shape
    V = embedding.shape[0]
    B = n * s
    tok = tokens.reshape(NW, B // NW // CHUNK, CHUNK).astype(jnp.int32)
    table128 = _tc_pack_table(V)(jnp.swapaxes(embedding, 0, 1))
    flat = _sc_gather(B, V)(tok, table128)
    out_col = _tc_transpose_out(n, s * DIM)(flat.reshape(n, s * DIM))
    return out_col.reshape(s, DIM, n).transpose(2, 0, 1)


# speedup vs baseline: 2.9020x; 1.1403x over previous
import functools

import jax
import jax.numpy as jnp
from jax import lax
from jax.experimental import pallas as pl
from jax.experimental.pallas import tpu as pltpu
from jax.experimental.pallas import tpu_sc as plsc

DIM = 64
NC = 2
NS = 16
NW = NC * NS
CHUNK = 128
NBUF = 2


@functools.cache
def _tc_pack_table(V: int):
    TB = 4096

    def body(in_ref, out_ref):
        out_ref[:, 0:DIM] = in_ref[...].T

    return pl.pallas_call(
        body,
        grid=((V + TB - 1) // TB,),
        in_specs=[pl.BlockSpec((DIM, TB), lambda i: (0, i))],
        out_specs=pl.BlockSpec((TB, 2 * DIM), lambda i: (i, 0)),
        out_shape=jax.ShapeDtypeStruct((V, 2 * DIM), jnp.float32),
    )


@functools.cache
def _tc_out_transpose(n: int, SD: int):
    NB = 128
    Q = SD // 128

    def body(in_ref, out_ref):
        x = in_ref[...].reshape(NB, Q, 128)
        for q in range(Q):
            out_ref[q] = x[:, q, :].T

    return pl.pallas_call(
        body,
        grid=(n // NB,),
        in_specs=[pl.BlockSpec((NB * Q, 128), lambda i: (i, 0))],
        out_specs=pl.BlockSpec((Q, 128, NB), lambda i: (0, 0, i)),
        out_shape=jax.ShapeDtypeStruct((Q, 128, n), jnp.float32),
    )


@functools.cache
def _sc_gather(B: int, V: int):
    bpw = B // NW
    nch = bpw // CHUNK
    ngrp = nch // (2 * NBUF)
    assert nch % (2 * NBUF) == 0
    mesh = plsc.VectorSubcoreMesh(core_axis_name="c", subcore_axis_name="s")

    @functools.partial(
        pl.kernel,
        mesh=mesh,
        out_type=jax.ShapeDtypeStruct((B, DIM), jnp.float32),
        scratch_types=[
            pltpu.VMEM((nch, CHUNK), jnp.int32),
            pltpu.VMEM((2 * NBUF, CHUNK, 2 * DIM), jnp.float32),
            pltpu.SemaphoreType.DMA,
            pltpu.SemaphoreType.DMA,
            pltpu.SemaphoreType.DMA,
            pltpu.SemaphoreType.DMA,
        ],
        compiler_params=pltpu.CompilerParams(use_tc_tiling_on_sc=False),
    )
    def emb(tok_hbm, table_hbm, out_hbm, idx_v, rows_v, g0, g1, w0, w1):
        wid = lax.axis_index("s") * NC + lax.axis_index("c")
        base_row = wid * bpw
        pltpu.sync_copy(tok_hbm.at[wid], idx_v)

        def fire_gather(chunk, buf, sem):
            pltpu.async_copy(table_hbm.at[idx_v.at[chunk]], rows_v.at[buf], sem)

        def fire_write(chunk, buf, sem):
            pltpu.async_copy(
                rows_v.at[buf, :, pl.ds(0, DIM)],
                out_hbm.at[pl.ds(base_row + chunk * CHUNK, CHUNK)],
                sem,
            )

        def drain_g(sem):
            pltpu.make_async_copy(
                table_hbm.at[pl.ds(0, CHUNK)], rows_v.at[0], sem
            ).wait()

        def drain_w(sem):
            pltpu.make_async_copy(
                rows_v.at[0, :, pl.ds(0, DIM)],
                out_hbm.at[pl.ds(base_row, CHUNK)],
                sem,
            ).wait()

        for b in range(NBUF):
            fire_gather(b, b, g0)

        def body(h, carry):
            base = h * (2 * NBUF)

            @pl.when(h > 0)
            def _():
                for b in range(NBUF):
                    drain_w(w1)

            for b in range(NBUF):
                fire_gather(base + NBUF + b, NBUF + b, g1)

            for b in range(NBUF):
                drain_g(g0)
            for b in range(NBUF):
                fire_write(base + b, b, w0)

            for b in range(NBUF):
                drain_w(w0)

            @pl.when(h + 1 < ngrp)
            def _():
                for b in range(NBUF):
                    fire_gather(base + 2 * NBUF + b, b, g0)

            for b in range(NBUF):
                drain_g(g1)
            for b in range(NBUF):
                fire_write(base + NBUF + b, NBUF + b, w1)
            return carry

        lax.fori_loop(0, ngrp, body, 0)
        for b in range(NBUF):
            drain_w(w1)

    return emb


def kernel(tokens, embedding):
    n, s = tokens.shape
    V = embedding.shape[0]
    B = n * s
    tok = tokens.reshape(NW, B // NW // CHUNK, CHUNK).astype(jnp.int32)
    table128 = _tc_pack_table(V)(jnp.swapaxes(embedding, 0, 1))
    flat = _sc_gather(B, V)(tok, table128)
    out3 = _tc_out_transpose(n, s * DIM)(flat.reshape(B // 2, 2 * DIM))
    return out3.reshape(s, DIM, n).transpose(2, 0, 1)
